# baseline (device time: 9399 ns/iter reference)
import jax
import jax.numpy as jnp
from jax import lax
from jax.experimental import pallas as pl
from jax.experimental.pallas import tpu as pltpu

N_DEV = 4
K = 8


def _peer_rdmas(my, acc_ref, comm_ref, send_sems, recv_sems, stage):
    rdmas = []
    for off in range(1, N_DEV):
        slot = N_DEV - 1 - off
        rdmas.append(
            pltpu.make_async_remote_copy(
                src_ref=acc_ref,
                dst_ref=comm_ref.at[stage, slot],
                send_sem=send_sems.at[stage, off - 1],
                recv_sem=recv_sems.at[stage, slot],
                device_id=((my + off) % N_DEV,),
                device_id_type=pl.DeviceIdType.MESH,
            )
        )
    return rdmas


def kernel(x):
    m_per, n = x.shape
    total_rows = N_DEV * m_per
    rows_blk = m_per // K

    def body(x_ref, out_ref, acc_a, acc_b, comm_ref, send_sems, recv_sems):
        k = pl.program_id(0)
        my = lax.axis_index("i")
        barrier_sem = pltpu.get_barrier_semaphore()

        @pl.when(k == 0)
        def _():
            for off in range(1, N_DEV):
                pl.semaphore_signal(
                    barrier_sem, inc=1,
                    device_id=((my + off) % N_DEV,),
                    device_id_type=pl.DeviceIdType.MESH,
                )
            acc_a[:, :] = jnp.sum(x_ref[:, :], axis=0, keepdims=True)

        @pl.when(jnp.logical_and(k > 0, k < K - 1))
        def _():
            acc_a[:, :] = acc_a[:, :] + jnp.sum(
                x_ref[:, :], axis=0, keepdims=True
            )

        @pl.when(k == K - 2)
        def _():
            pl.semaphore_wait(barrier_sem, N_DEV - 1)
            for rdma in _peer_rdmas(my, acc_a, comm_ref, send_sems,
                                    recv_sems, 0):
                rdma.start()

        @pl.when(k == K - 1)
        def _():
            acc_b[:, :] = jnp.sum(x_ref[:, :], axis=0, keepdims=True)
            rdmas_b = _peer_rdmas(my, acc_b, comm_ref, send_sems,
                                  recv_sems, 1)
            for rdma in rdmas_b:
                rdma.start()
            rdmas_a = _peer_rdmas(my, acc_a, comm_ref, send_sems,
                                  recv_sems, 0)
            for rdma in rdmas_a + rdmas_b:
                rdma.wait()

            acc = acc_a[:, :] + acc_b[:, :]
            for stage in range(2):
                for slot in range(N_DEV - 1):
                    acc = acc + comm_ref[stage, slot, :, :]
            out_ref[:, :] = acc * (1.0 / total_rows)

    return pl.pallas_call(
        body,
        grid=(K,),
        out_shape=jax.ShapeDtypeStruct((1, n), x.dtype),
        in_specs=[
            pl.BlockSpec((rows_blk, n), lambda k: (k, 0), memory_space=pltpu.VMEM)
        ],
        out_specs=pl.BlockSpec((1, n), lambda k: (0, 0), memory_space=pltpu.VMEM),
        scratch_shapes=[
            pltpu.VMEM((1, n), x.dtype),
            pltpu.VMEM((1, n), x.dtype),
            pltpu.VMEM((2, N_DEV - 1, 1, n), x.dtype),
            pltpu.SemaphoreType.DMA((2, N_DEV - 1)),
            pltpu.SemaphoreType.DMA((2, N_DEV - 1)),
        ],
        compiler_params=pltpu.CompilerParams(collective_id=0),
    )(x)
